# baseline (speedup 1.0000x reference)
import jax
import jax.numpy as jnp
from jax.experimental import pallas as pl
from jax.experimental.pallas import tpu as pltpu

N_REF = 8192
N_SRC = 8192
FEAT = 64
K = 256
ROW_BLOCK = 256
NUM_ROW_BLOCKS = N_REF // ROW_BLOCK
CHUNK = 256
NUM_CHUNKS = N_REF // CHUNK

_DOT_DIMS = (((1,), (0,)), ((), ()))


def _rowmax_kernel(ref_ref, srcT_ref, out_ref):
    sim = jax.lax.dot_general(
        ref_ref[...], srcT_ref[...], _DOT_DIMS,
        preferred_element_type=jnp.float32)
    out_ref[...] = jnp.max(sim, axis=1).reshape(1, 1, ROW_BLOCK)


def _cumsum_lanes(x):
    n = x.shape[1]
    shift = 1
    while shift < n:
        x = x + jnp.concatenate(
            [jnp.zeros((1, shift), x.dtype), x[:, :-shift]], axis=1)
        shift *= 2
    return x


def _select_kernel(ref_ref, srcT_ref, rowmax_ref,
                   rows_ref, cols_ref, scores_ref, stripe):
    row_iota = jax.lax.broadcasted_iota(jnp.int32, (1, N_REF), 1)
    col_iota = jax.lax.broadcasted_iota(jnp.int32, (1, N_SRC), 1)
    out_iota = jax.lax.broadcasted_iota(jnp.int32, (1, K), 1)
    slot_iota = jax.lax.broadcasted_iota(jnp.int32, (1, K), 1)
    slot_col_iota = jax.lax.broadcasted_iota(jnp.int32, (K, CHUNK), 0)

    m = rowmax_ref[...]
    ib = jax.lax.bitcast_convert_type(m, jnp.int32)
    key = jnp.where(ib < 0, ib ^ jnp.int32(0x7FFFFFFF), ib)

    npos = jnp.sum((key >= 0).astype(jnp.int32))
    lo0 = jnp.where(npos >= K, jnp.int32(0), jnp.int32(-2**31))
    hi0 = jnp.where(npos >= K, jnp.int32(2**31 - 1), jnp.int32(-1))

    def bisect(_, lh):
        lo, hi = lh
        span = hi - lo
        mid = lo + span // 2 + span % 2
        ok = jnp.sum((key >= mid).astype(jnp.int32)) >= K
        return jnp.where(ok, mid, lo), jnp.where(ok, hi, mid - 1)

    kstar, _ = jax.lax.fori_loop(0, 32, bisect, (lo0, hi0))

    gt = key > kstar
    n_gt = jnp.sum(gt.astype(jnp.int32))
    tie = key == kstar
    tie_rank = _cumsum_lanes(tie.astype(jnp.int32))
    sel = gt | (tie & (tie_rank <= K - n_gt))
    ranks = _cumsum_lanes(sel.astype(jnp.int32))

    gathered = jnp.zeros((K, FEAT), jnp.float32)
    rowid = jnp.zeros((K, 1), jnp.float32)
    for c in range(NUM_CHUNKS):
        sl = slice(c * CHUNK, (c + 1) * CHUNK)
        onehot = (jnp.broadcast_to(ranks[:, sl], (K, CHUNK)) ==
                  slot_col_iota + 1) & jnp.broadcast_to(sel[:, sl], (K, CHUNK))
        onehot = onehot.astype(jnp.float32)
        gathered = gathered + jax.lax.dot_general(
            onehot, ref_ref[sl, :], _DOT_DIMS,
            preferred_element_type=jnp.float32)
        rowid = rowid + jnp.sum(
            onehot * row_iota[:, sl].astype(jnp.float32),
            axis=1, keepdims=True)
    rowid1 = rowid.reshape(1, K)

    sim = jax.lax.dot_general(
        gathered, srcT_ref[...], _DOT_DIMS,
        preferred_element_type=jnp.float32)
    stripe[...] = jnp.exp(-(2.0 - 2.0 * sim))

    colb = jax.lax.broadcasted_iota(jnp.int32, (K, N_SRC), 1)
    lv, lc = [], []
    for t in range(4):
        ev = stripe[...]
        if t == 0:
            elig_v = ev
        else:
            pv, pc = lv[-1], lc[-1]
            elig = (ev < pv) | ((ev == pv) & (colb > pc))
            elig_v = jnp.where(elig, ev, -jnp.inf)
        vk = jnp.max(elig_v, axis=1, keepdims=True)
        ck = jnp.min(jnp.where(elig_v == vk, colb, jnp.int32(N_SRC)),
                     axis=1, keepdims=True)
        lv.append(vk)
        lc.append(ck)
    lv1, lv2, lv3, lv4 = [v.reshape(1, K) for v in lv]
    lc1, lc2, lc3, lc4 = [c.reshape(1, K) for c in lc]

    rows_ref[...] = jnp.zeros((1, K), jnp.int32)
    cols_ref[...] = jnp.zeros((1, K), jnp.int32)
    scores_ref[...] = jnp.zeros((1, K), jnp.float32)

    def step(r, carry):
        heads, heads_col, ptr = carry
        best = jnp.max(heads)
        i_star = jnp.min(jnp.where(heads == best, slot_iota, jnp.int32(K)))
        sel1 = slot_iota == i_star
        r_em = jnp.max(jnp.where(sel1, rowid1, -1.0)).astype(jnp.int32)
        best_col = jnp.max(jnp.where(sel1, heads_col, jnp.int32(-1)))
        p = jnp.max(jnp.where(sel1, ptr, jnp.int32(-1)))

        def next_from_lists(_):
            nhv = jnp.where(ptr == 0, lv2, jnp.where(ptr == 1, lv3, lv4))
            ncv = jnp.where(ptr == 0, lc2, jnp.where(ptr == 1, lc3, lc4))
            nh = jnp.max(jnp.where(sel1, nhv, -jnp.inf))
            nc = jnp.max(jnp.where(sel1, ncv, jnp.int32(-1)))
            return nh, nc

        def next_from_rescan(_):
            e_row = stripe[pl.ds(i_star, 1), :]
            elig = (e_row < best) | ((e_row == best) & (col_iota > best_col))
            ev = jnp.where(elig, e_row, -jnp.inf)
            nh = jnp.max(ev)
            nc = jnp.min(jnp.where(ev == nh, col_iota, jnp.int32(N_SRC)))
            return nh, nc

        nh, nc = jax.lax.cond(p >= 3, next_from_rescan, next_from_lists, 0)

        rows_ref[...] = jnp.where(out_iota == r, r_em, rows_ref[...])
        cols_ref[...] = jnp.where(out_iota == r, best_col, cols_ref[...])
        scores_ref[...] = jnp.where(out_iota == r, best, scores_ref[...])
        return (jnp.where(sel1, nh, heads),
                jnp.where(sel1, nc, heads_col),
                jnp.where(sel1, ptr + 1, ptr))

    jax.lax.fori_loop(
        0, K, step,
        (lv1, lc1, jnp.zeros((1, K), jnp.int32)))


@jax.jit
def kernel(ref_feats, src_feats):
    srcT = src_feats.T

    rowmax = pl.pallas_call(
        _rowmax_kernel,
        grid=(NUM_ROW_BLOCKS,),
        in_specs=[
            pl.BlockSpec((ROW_BLOCK, FEAT), lambda b: (b, 0)),
            pl.BlockSpec((FEAT, N_SRC), lambda b: (0, 0)),
        ],
        out_specs=pl.BlockSpec((1, 1, ROW_BLOCK), lambda b: (b, 0, 0)),
        out_shape=jax.ShapeDtypeStruct((NUM_ROW_BLOCKS, 1, ROW_BLOCK),
                                       jnp.float32),
    )(ref_feats, srcT)

    rows, cols, scores = pl.pallas_call(
        _select_kernel,
        in_specs=[
            pl.BlockSpec((N_REF, FEAT), lambda: (0, 0)),
            pl.BlockSpec((FEAT, N_SRC), lambda: (0, 0)),
            pl.BlockSpec((1, N_REF), lambda: (0, 0)),
        ],
        out_specs=[
            pl.BlockSpec((1, K), lambda: (0, 0)),
            pl.BlockSpec((1, K), lambda: (0, 0)),
            pl.BlockSpec((1, K), lambda: (0, 0)),
        ],
        out_shape=[
            jax.ShapeDtypeStruct((1, K), jnp.int32),
            jax.ShapeDtypeStruct((1, K), jnp.int32),
            jax.ShapeDtypeStruct((1, K), jnp.float32),
        ],
        scratch_shapes=[
            pltpu.VMEM((K, N_SRC), jnp.float32),
        ],
    )(ref_feats, srcT, rowmax.reshape(1, N_REF))

    return rows.reshape(K), cols.reshape(K), scores.reshape(K)


# speedup vs baseline: 1.1426x; 1.1426x over previous
import jax
import jax.numpy as jnp
from jax.experimental import pallas as pl
from jax.experimental.pallas import tpu as pltpu

N_REF = 8192
N_SRC = 8192
FEAT = 64
K = 256
ROW_BLOCK = 256
NUM_ROW_BLOCKS = N_REF // ROW_BLOCK
CHUNK = 256
NUM_CHUNKS = N_REF // CHUNK

_DOT_DIMS = (((1,), (0,)), ((), ()))


def _rowmax_kernel(ref_ref, srcT_ref, out_ref):
    sim = jax.lax.dot_general(
        ref_ref[...], srcT_ref[...], _DOT_DIMS,
        preferred_element_type=jnp.float32)
    out_ref[...] = jnp.max(sim, axis=1).reshape(1, 1, ROW_BLOCK)


def _cumsum_lanes(x):
    n = x.shape[1]
    shift = 1
    while shift < n:
        x = x + jnp.concatenate(
            [jnp.zeros((1, shift), x.dtype), x[:, :-shift]], axis=1)
        shift *= 2
    return x


def _select_kernel(ref_ref, srcT_ref, rowmax_ref,
                   rows_ref, cols_ref, scores_ref, stripe):
    row_iota = jax.lax.broadcasted_iota(jnp.int32, (1, N_REF), 1)
    col_iota = jax.lax.broadcasted_iota(jnp.int32, (1, N_SRC), 1)
    out_iota = jax.lax.broadcasted_iota(jnp.int32, (1, K), 1)
    slot_iota = jax.lax.broadcasted_iota(jnp.int32, (1, K), 1)
    slot_col_iota = jax.lax.broadcasted_iota(jnp.int32, (K, CHUNK), 0)

    m = rowmax_ref[...]
    ib = jax.lax.bitcast_convert_type(m, jnp.int32)
    key = jnp.where(ib < 0, ib ^ jnp.int32(0x7FFFFFFF), ib)

    npos = jnp.sum((key >= 0).astype(jnp.int32))
    lo0 = jnp.where(npos >= K, jnp.int32(0), jnp.int32(-2**31))
    hi0 = jnp.where(npos >= K, jnp.int32(2**31 - 1), jnp.int32(-1))

    def bisect(_, lh):
        lo, hi = lh
        span = hi - lo
        mid = lo + span // 2 + span % 2
        ok = jnp.sum((key >= mid).astype(jnp.int32)) >= K
        return jnp.where(ok, mid, lo), jnp.where(ok, hi, mid - 1)

    kstar, _ = jax.lax.fori_loop(0, 32, bisect, (lo0, hi0))

    gt = key > kstar
    n_gt = jnp.sum(gt.astype(jnp.int32))
    tie = key == kstar
    tie_rank = _cumsum_lanes(tie.astype(jnp.int32))
    sel = gt | (tie & (tie_rank <= K - n_gt))
    ranks = _cumsum_lanes(sel.astype(jnp.int32))

    gathered = jnp.zeros((K, FEAT), jnp.float32)
    rowid = jnp.zeros((K, 1), jnp.float32)
    for c in range(NUM_CHUNKS):
        sl = slice(c * CHUNK, (c + 1) * CHUNK)
        onehot = (jnp.broadcast_to(ranks[:, sl], (K, CHUNK)) ==
                  slot_col_iota + 1) & jnp.broadcast_to(sel[:, sl], (K, CHUNK))
        onehot = onehot.astype(jnp.float32)
        gathered = gathered + jax.lax.dot_general(
            onehot, ref_ref[sl, :], _DOT_DIMS,
            preferred_element_type=jnp.float32)
        rowid = rowid + jnp.sum(
            onehot * row_iota[:, sl].astype(jnp.float32),
            axis=1, keepdims=True)
    rowid1 = rowid.reshape(1, K)

    sim = jax.lax.dot_general(
        gathered, srcT_ref[...], _DOT_DIMS,
        preferred_element_type=jnp.float32)
    stripe[...] = jnp.exp(-(2.0 - 2.0 * sim))

    colb = jax.lax.broadcasted_iota(jnp.int32, (K, N_SRC), 1)
    lv, lc, lv_r, lc_r = [], [], [], []
    for t in range(4):
        ev = stripe[...]
        if t == 0:
            elig_v = ev
        else:
            elig = (ev < lv[-1]) | ((ev == lv[-1]) & (colb > lc[-1]))
            elig_v = jnp.where(elig, ev, -jnp.inf)
        vk = jnp.max(elig_v, axis=1, keepdims=True)
        ck = jnp.min(jnp.where(elig_v == vk, colb, jnp.int32(N_SRC)),
                     axis=1, keepdims=True)
        lv.append(vk)
        lc.append(ck)
        lv_r.append(vk.reshape(1, K))
        lc_r.append(ck.reshape(1, K))

    rowid_col = rowid.astype(jnp.int32)
    rid_row = rowid1.astype(jnp.int32)
    f_col = [rowid_col * jnp.int32(N_SRC) + c for c in lc]
    f_row = [rid_row * jnp.int32(N_SRC) + c for c in lc_r]
    slot_b = jax.lax.broadcasted_iota(jnp.int32, (K, K), 1)

    rank_col = []
    for ti in range(4):
        acc = jnp.zeros((K, 1), jnp.int32)
        for tj in range(4):
            prec = ((lv_r[tj] > lv[ti]) |
                    ((lv_r[tj] == lv[ti]) & (f_row[tj] < f_col[ti])))
            acc = acc + jnp.sum(prec.astype(jnp.int32), axis=1, keepdims=True)
        rank_col.append(acc)

    deep = jnp.min(rank_col[3]) < jnp.int32(K - 1)

    @pl.when(jnp.logical_not(deep))
    def _fast():
        svals = jnp.zeros((1, K), jnp.float32)
        scols = jnp.zeros((1, K), jnp.float32)
        oh_sum = jnp.zeros((K, K), jnp.float32)
        for t in range(4):
            onehot = (jnp.broadcast_to(rank_col[t], (K, K)) == slot_b)
            onehot = onehot.astype(jnp.float32)
            oh_sum = oh_sum + onehot
            svals = svals + jax.lax.dot_general(
                lv_r[t], onehot, _DOT_DIMS,
                preferred_element_type=jnp.float32)
            scols = scols + jax.lax.dot_general(
                lc_r[t].astype(jnp.float32), onehot, _DOT_DIMS,
                preferred_element_type=jnp.float32)
        srows = jax.lax.dot_general(rowid1, oh_sum, _DOT_DIMS,
                                    preferred_element_type=jnp.float32)
        rows_ref[...] = srows.astype(jnp.int32)
        cols_ref[...] = scols.astype(jnp.int32)
        scores_ref[...] = svals

    @pl.when(deep)
    def _slow():
        heads0 = jnp.max(stripe[...], axis=1).reshape(1, K)
        rows_ref[...] = jnp.zeros((1, K), jnp.int32)
        cols_ref[...] = jnp.zeros((1, K), jnp.int32)
        scores_ref[...] = jnp.zeros((1, K), jnp.float32)

        def step(r, heads):
            best = jnp.max(heads)
            i_star = jnp.min(jnp.where(heads == best, slot_iota, jnp.int32(K)))
            sel1 = slot_iota == i_star
            r_em = jnp.max(jnp.where(sel1, rowid1, -1.0)).astype(jnp.int32)

            e_row = stripe[pl.ds(i_star, 1), :]
            hit = e_row == best
            best_col = jnp.min(jnp.where(hit, col_iota, jnp.int32(N_SRC)))
            e_next = jnp.where(col_iota == best_col, -jnp.inf, e_row)
            stripe[pl.ds(i_star, 1), :] = e_next
            nh = jnp.max(e_next)

            rows_ref[...] = jnp.where(out_iota == r, r_em, rows_ref[...])
            cols_ref[...] = jnp.where(out_iota == r, best_col, cols_ref[...])
            scores_ref[...] = jnp.where(out_iota == r, best, scores_ref[...])
            return jnp.where(sel1, nh, heads)

        jax.lax.fori_loop(0, K, step, heads0)


@jax.jit
def kernel(ref_feats, src_feats):
    srcT = src_feats.T

    rowmax = pl.pallas_call(
        _rowmax_kernel,
        grid=(NUM_ROW_BLOCKS,),
        in_specs=[
            pl.BlockSpec((ROW_BLOCK, FEAT), lambda b: (b, 0)),
            pl.BlockSpec((FEAT, N_SRC), lambda b: (0, 0)),
        ],
        out_specs=pl.BlockSpec((1, 1, ROW_BLOCK), lambda b: (b, 0, 0)),
        out_shape=jax.ShapeDtypeStruct((NUM_ROW_BLOCKS, 1, ROW_BLOCK),
                                       jnp.float32),
    )(ref_feats, srcT)

    rows, cols, scores = pl.pallas_call(
        _select_kernel,
        in_specs=[
            pl.BlockSpec((N_REF, FEAT), lambda: (0, 0)),
            pl.BlockSpec((FEAT, N_SRC), lambda: (0, 0)),
            pl.BlockSpec((1, N_REF), lambda: (0, 0)),
        ],
        out_specs=[
            pl.BlockSpec((1, K), lambda: (0, 0)),
            pl.BlockSpec((1, K), lambda: (0, 0)),
            pl.BlockSpec((1, K), lambda: (0, 0)),
        ],
        out_shape=[
            jax.ShapeDtypeStruct((1, K), jnp.int32),
            jax.ShapeDtypeStruct((1, K), jnp.int32),
            jax.ShapeDtypeStruct((1, K), jnp.float32),
        ],
        scratch_shapes=[
            pltpu.VMEM((K, N_SRC), jnp.float32),
        ],
    )(ref_feats, srcT, rowmax.reshape(1, N_REF))

    return rows.reshape(K), cols.reshape(K), scores.reshape(K)
